# baseline (speedup 1.0000x reference)
import jax
import jax.numpy as jnp
from jax import lax
from jax.experimental import pallas as pl
from jax.experimental.pallas import tpu as pltpu
from jax.experimental.pallas import tpu_sc as plsc

N = 100000
HALF = 128
OUT_D = 256
NC = 2
NS = 16
NW = NC * NS
C = 160
BUFS = 3
NCHUNK = N // C
J = -(-NCHUNK // NW)
LAST_FULL = NCHUNK - (J - 1) * NW


def _pe_kernel(node_hbm, time_hbm, tpe_hbm, spe_hbm, out_hbm, *scratch):
    nidx = scratch[0:BUFS]
    tidx = scratch[BUFS:2 * BUFS]
    rows = scratch[2 * BUFS:3 * BUFS]
    gt = scratch[3 * BUFS:4 * BUFS]
    gs = scratch[4 * BUFS:5 * BUFS]
    wb = scratch[5 * BUFS:6 * BUFS]
    it = scratch[6 * BUFS:7 * BUFS]
    inm = scratch[7 * BUFS:8 * BUFS]

    wid = lax.axis_index("s") * NC + lax.axis_index("c")
    has_last = wid < LAST_FULL

    def descs(j):
        b = j % BUFS
        base = (wid + j * NW) * C
        return (
            pltpu.make_async_copy(
                tpe_hbm.at[tidx[b]], rows[b].at[:, pl.ds(0, HALF)], gt[b]),
            pltpu.make_async_copy(
                spe_hbm.at[nidx[b]], rows[b].at[:, pl.ds(HALF, HALF)], gs[b]),
            pltpu.make_async_copy(
                rows[b], out_hbm.at[pl.ds(base, C)], wb[b]),
            pltpu.make_async_copy(
                time_hbm.at[pl.ds(base, C)], tidx[b], it[b]),
            pltpu.make_async_copy(
                node_hbm.at[pl.ds(base, C)], nidx[b], inm[b]),
        )

    d = [descs(j) for j in range(J)]

    def prefetch_idx(j):
        d[j][3].start()
        d[j][4].start()

    def issue_gathers(j):
        d[j][3].wait()
        d[j][4].wait()
        d[j][0].start()
        d[j][1].start()

    prefetch_idx(0)
    prefetch_idx(1)
    issue_gathers(0)
    for j in range(J):
        if j + 1 < J:
            def lookahead(jj=j):
                if jj + 1 >= BUFS:
                    d[jj + 1 - BUFS][2].wait()
                issue_gathers(jj + 1)
                if jj + 2 < J - 1:
                    prefetch_idx(jj + 2)
                elif jj + 2 == J - 1:
                    pl.when(has_last)(lambda: prefetch_idx(jj + 2))
            if j + 1 == J - 1:
                pl.when(has_last)(lookahead)
            else:
                lookahead()

        def finish(jj=j):
            d[jj][0].wait()
            d[jj][1].wait()
            d[jj][2].start()
        if j == J - 1:
            pl.when(has_last)(finish)
        else:
            finish()

    for k in range(1, BUFS + 1):
        d[J - k][2].wait()


def kernel(x, node_ids, time_ids, temporal_pe, spatial_pe):
    del x
    mesh = plsc.VectorSubcoreMesh(core_axis_name="c", subcore_axis_name="s")
    f = pl.kernel(
        _pe_kernel,
        out_type=jax.ShapeDtypeStruct((N, OUT_D), jnp.float32),
        mesh=mesh,
        scratch_types=(
            [pltpu.VMEM((C,), jnp.int32) for _ in range(2 * BUFS)]
            + [pltpu.VMEM((C, OUT_D), jnp.float32) for _ in range(BUFS)]
            + [pltpu.SemaphoreType.DMA for _ in range(5 * BUFS)]
        ),
    )
    return f(node_ids, time_ids, temporal_pe, spatial_pe)


# speedup vs baseline: 1.0352x; 1.0352x over previous
import jax
import jax.numpy as jnp
from jax import lax
from jax.experimental import pallas as pl
from jax.experimental.pallas import tpu as pltpu
from jax.experimental.pallas import tpu_sc as plsc

N = 100000
HALF = 128
OUT_D = 256
NC = 2
NS = 16
NW = NC * NS
C = 160
BUFS = 3
NCHUNK = N // C
J = -(-NCHUNK // NW)
LAST_FULL = NCHUNK - (J - 1) * NW


def _pe_kernel(node_hbm, time_hbm, tpe_hbm, spe_hbm, out_hbm, *scratch):
    nidx = scratch[0:BUFS]
    tidx = scratch[BUFS:2 * BUFS]
    trows = scratch[2 * BUFS:3 * BUFS]
    srows = scratch[3 * BUFS:4 * BUFS]
    gt = scratch[4 * BUFS:5 * BUFS]
    gs = scratch[5 * BUFS:6 * BUFS]
    wt = scratch[6 * BUFS:7 * BUFS]
    ws = scratch[7 * BUFS:8 * BUFS]
    it = scratch[8 * BUFS:9 * BUFS]
    inm = scratch[9 * BUFS:10 * BUFS]

    wid = lax.axis_index("s") * NC + lax.axis_index("c")
    has_last = wid < LAST_FULL

    def descs(j):
        b = j % BUFS
        base = (wid + j * NW) * C
        return (
            pltpu.make_async_copy(tpe_hbm.at[tidx[b]], trows[b], gt[b]),
            pltpu.make_async_copy(spe_hbm.at[nidx[b]], srows[b], gs[b]),
            pltpu.make_async_copy(
                trows[b], out_hbm.at[pl.ds(base, C), pl.ds(0, HALF)], wt[b]),
            pltpu.make_async_copy(
                srows[b], out_hbm.at[pl.ds(base, C), pl.ds(HALF, HALF)], ws[b]),
            pltpu.make_async_copy(
                time_hbm.at[pl.ds(base, C)], tidx[b], it[b]),
            pltpu.make_async_copy(
                node_hbm.at[pl.ds(base, C)], nidx[b], inm[b]),
        )

    d = [descs(j) for j in range(J)]

    def prefetch_idx(j):
        d[j][4].start()
        d[j][5].start()

    def issue_gathers(j):
        d[j][4].wait()
        d[j][5].wait()
        d[j][0].start()
        d[j][1].start()

    prefetch_idx(0)
    prefetch_idx(1)
    issue_gathers(0)
    for j in range(J):
        if j + 1 < J:
            def lookahead(jj=j):
                if jj + 1 >= BUFS:
                    d[jj + 1 - BUFS][2].wait()
                    d[jj + 1 - BUFS][3].wait()
                issue_gathers(jj + 1)
                if jj + 2 < J - 1:
                    prefetch_idx(jj + 2)
                elif jj + 2 == J - 1:
                    pl.when(has_last)(lambda: prefetch_idx(jj + 2))
            if j + 1 == J - 1:
                pl.when(has_last)(lookahead)
            else:
                lookahead()

        def finish(jj=j):
            d[jj][0].wait()
            d[jj][1].wait()
            d[jj][2].start()
            d[jj][3].start()
        if j == J - 1:
            pl.when(has_last)(finish)
        else:
            finish()

    for k in range(1, BUFS + 1):
        d[J - k][2].wait()
        d[J - k][3].wait()


def kernel(x, node_ids, time_ids, temporal_pe, spatial_pe):
    del x
    mesh = plsc.VectorSubcoreMesh(core_axis_name="c", subcore_axis_name="s")
    f = pl.kernel(
        _pe_kernel,
        out_type=jax.ShapeDtypeStruct((N, OUT_D), jnp.float32),
        mesh=mesh,
        scratch_types=(
            [pltpu.VMEM((C,), jnp.int32) for _ in range(2 * BUFS)]
            + [pltpu.VMEM((C, HALF), jnp.float32) for _ in range(2 * BUFS)]
            + [pltpu.SemaphoreType.DMA for _ in range(6 * BUFS)]
        ),
    )
    return f(node_ids, time_ids, temporal_pe, spatial_pe)


# speedup vs baseline: 1.5384x; 1.4861x over previous
import jax
import jax.numpy as jnp
from jax import lax
from jax.experimental import pallas as pl
from jax.experimental.pallas import tpu as pltpu
from jax.experimental.pallas import tpu_sc as plsc

N = 100000
HALF = 128
OUT_D = 256
T = 500
NC = 2
NS = 16
NW = NC * NS
C = 160
BUFS = 3
NCHUNK = N // C
J = -(-NCHUNK // NW)
LAST_FULL = NCHUNK - (J - 1) * NW


def _pe_kernel(node_hbm, time_hbm, tpe_hbm, spe_hbm, out_hbm, *scratch):
    nidx = scratch[0:BUFS]
    tidx = scratch[BUFS:2 * BUFS]
    trows = scratch[2 * BUFS:3 * BUFS]
    srows = scratch[3 * BUFS:4 * BUFS]
    gt = scratch[4 * BUFS:5 * BUFS]
    gs = scratch[5 * BUFS:6 * BUFS]
    wt = scratch[6 * BUFS:7 * BUFS]
    ws = scratch[7 * BUFS:8 * BUFS]
    it = scratch[8 * BUFS:9 * BUFS]
    inm = scratch[9 * BUFS:10 * BUFS]
    tbl = scratch[10 * BUFS]
    tsem = scratch[10 * BUFS + 1]

    wid = lax.axis_index("s") * NC + lax.axis_index("c")
    has_last = wid < LAST_FULL

    @pl.when(lax.axis_index("s") == 0)
    def _stage_tbl():
        cp = pltpu.make_async_copy(tpe_hbm, tbl, tsem)
        cp.start()
        cp.wait()

    def descs(j):
        b = j % BUFS
        base = (wid + j * NW) * C
        return (
            pltpu.make_async_copy(tbl.at[tidx[b]], trows[b], gt[b]),
            pltpu.make_async_copy(spe_hbm.at[nidx[b]], srows[b], gs[b]),
            pltpu.make_async_copy(
                trows[b], out_hbm.at[pl.ds(base, C), pl.ds(0, HALF)], wt[b]),
            pltpu.make_async_copy(
                srows[b], out_hbm.at[pl.ds(base, C), pl.ds(HALF, HALF)], ws[b]),
            pltpu.make_async_copy(
                time_hbm.at[pl.ds(base, C)], tidx[b], it[b]),
            pltpu.make_async_copy(
                node_hbm.at[pl.ds(base, C)], nidx[b], inm[b]),
        )

    d = [descs(j) for j in range(J)]

    def prefetch_idx(j):
        d[j][4].start()
        d[j][5].start()

    def issue_gathers(j):
        d[j][4].wait()
        d[j][5].wait()
        d[j][0].start()
        d[j][1].start()

    prefetch_idx(0)
    prefetch_idx(1)
    plsc.subcore_barrier()
    issue_gathers(0)
    for j in range(J):
        if j + 1 < J:
            def lookahead(jj=j):
                if jj + 1 >= BUFS:
                    d[jj + 1 - BUFS][2].wait()
                    d[jj + 1 - BUFS][3].wait()
                issue_gathers(jj + 1)
                if jj + 2 < J - 1:
                    prefetch_idx(jj + 2)
                elif jj + 2 == J - 1:
                    pl.when(has_last)(lambda: prefetch_idx(jj + 2))
            if j + 1 == J - 1:
                pl.when(has_last)(lookahead)
            else:
                lookahead()

        def finish(jj=j):
            d[jj][0].wait()
            d[jj][1].wait()
            d[jj][2].start()
            d[jj][3].start()
        if j == J - 1:
            pl.when(has_last)(finish)
        else:
            finish()

    for k in range(1, BUFS + 1):
        d[J - k][2].wait()
        d[J - k][3].wait()


def kernel(x, node_ids, time_ids, temporal_pe, spatial_pe):
    del x
    mesh = plsc.VectorSubcoreMesh(core_axis_name="c", subcore_axis_name="s")
    f = pl.kernel(
        _pe_kernel,
        out_type=jax.ShapeDtypeStruct((N, OUT_D), jnp.float32),
        mesh=mesh,
        scratch_types=(
            [pltpu.VMEM((C,), jnp.int32) for _ in range(2 * BUFS)]
            + [pltpu.VMEM((C, HALF), jnp.float32) for _ in range(2 * BUFS)]
            + [pltpu.SemaphoreType.DMA for _ in range(6 * BUFS)]
            + [pltpu.VMEM_SHARED((T, HALF), jnp.float32),
               pltpu.SemaphoreType.DMA]
        ),
    )
    return f(node_ids, time_ids, temporal_pe, spatial_pe)
